# baseline (speedup 1.0000x reference)
import functools

import jax
import jax.numpy as jnp
from jax import lax
from jax.experimental import pallas as pl
from jax.experimental.pallas import tpu as pltpu
from jax.experimental.pallas import tpu_sc as plsc

NC = 2
NS = 16
NW = NC * NS
L = 16

BLK = 128


@functools.lru_cache(maxsize=None)
def _make_embed(S, BT, D):
    assert BT % (NW * BLK) == 0 and S % 8 == 0
    jpw = BT // (NW * BLK)
    bpw = jpw * BLK
    nk = S // 8
    upk = 8 * jpw
    hshift = (D - 1).bit_length()

    mesh = plsc.VectorSubcoreMesh(
        core_axis_name="c", subcore_axis_name="s",
        num_cores=NC, num_subcores=NS)

    @functools.partial(
        pl.kernel,
        mesh=mesh,
        out_type=jax.ShapeDtypeStruct((S, D, BT), jnp.float32),
        scratch_types=[
            pltpu.VMEM((8, bpw), jnp.int32),
            pltpu.VMEM((2, BLK), jnp.int32),
            pltpu.VMEM((2, BLK, 2 * D), jnp.float32),
            pltpu.VMEM((2, D, BLK), jnp.float32),
            pltpu.SemaphoreType.DMA,
            pltpu.SemaphoreType.DMA,
            pltpu.SemaphoreType.DMA,
            pltpu.SemaphoreType.DMA,
        ],
        compiler_params=pltpu.CompilerParams(
            use_tc_tiling_on_sc=True, needs_layout_passes=False),
    )
    def embed_kernel(tab_hbm, ids_hbm, out_hbm, ids_v, kidx_v, g_v, t_v,
                     gsem0, gsem1, osem0, osem1):
        gsem = (gsem0, gsem1)
        osem = (osem0, osem1)
        wid = lax.axis_index("s") * NC + lax.axis_index("c")
        b0 = wid * bpw
        iota16 = lax.iota(jnp.int32, L)
        zeros16 = iota16 * 0

        def fire_gather(u, q):
            r = u // jpw
            base = (u % jpw) * BLK
            for cc in range(BLK // L):
                ids16 = ids_v[r, pl.ds(base + cc * L, L)]
                kidx_v[q, pl.ds(cc * L, L)] = ids16 >> 1
            pltpu.async_copy(
                tab_hbm.at[kidx_v.at[q]], g_v.at[q], gsem[q])

        def gather_wait(q):
            pltpu.make_async_copy(
                tab_hbm.at[kidx_v.at[q]], g_v.at[q], gsem[q]).wait()

        def out_wait(q):
            pltpu.make_async_copy(
                t_v.at[q],
                out_hbm.at[0, pl.ds(0, D), pl.ds(0, BLK)], osem[q]).wait()

        def do_unit(k, u, p):
            s0 = k * 8
            r = u // jpw
            base = (u % jpw) * BLK

            @pl.when(u + 1 < upk)
            def _():
                fire_gather(u + 1, 1 - p)

            gather_wait(p)

            @pl.when(k * upk + u >= 2)
            def _():
                out_wait(p)

            gp = g_v.at[p]
            for jj in range(BLK // L):
                ids16 = ids_v[r, pl.ds(base + jj * L, L)]
                col0 = (ids16 & 1) << hshift
                rows = iota16 + (jj * L)
                scol = iota16 + (jj * L)

                def drow(d16, _, col0=col0, rows=rows, scol=scol):
                    for d1 in range(16):
                        dd = d16 * 16 + d1
                        vals = plsc.load_gather(gp, [rows, col0 + dd])
                        plsc.store_scatter(
                            t_v.at[p], [zeros16 + dd, scol], vals)
                    return 0

                lax.fori_loop(0, D // 16, drow, 0)

            pltpu.async_copy(
                t_v.at[p],
                out_hbm.at[s0 + r, pl.ds(0, D), pl.ds(b0 + base, BLK)],
                osem[p])

        def kblock(k, _):
            pltpu.sync_copy(
                ids_hbm.at[pl.ds(k * 8, 8), pl.ds(b0, bpw)], ids_v)
            fire_gather(0, 0)

            def pair(g, _):
                do_unit(k, 2 * g, 0)
                do_unit(k, 2 * g + 1, 1)
                return 0

            lax.fori_loop(0, upk // 2, pair, 0)
            return 0

        lax.fori_loop(0, nk, kblock, 0)
        out_wait(0)
        out_wait(1)

    return embed_kernel


def kernel(token_ids, weight):
    BT, S = token_ids.shape
    V, D = weight.shape
    ids_t = token_ids.T.astype(jnp.int32)
    tab2 = weight.reshape(V // 2, 2 * D)
    out3 = _make_embed(S, BT, D)(tab2, ids_t)
    return out3.transpose(2, 0, 1)


# speedup vs baseline: 1.8620x; 1.8620x over previous
import functools

import jax
import jax.numpy as jnp
from jax import lax
from jax.experimental import pallas as pl
from jax.experimental.pallas import tpu as pltpu
from jax.experimental.pallas import tpu_sc as plsc

NC = 2
NS = 16
NW = NC * NS
L = 16

BLK = 128


@functools.lru_cache(maxsize=None)
def _make_embed(S, BT, D):
    assert BT % (NW * BLK) == 0 and S % 8 == 0
    jpw = BT // (NW * BLK)
    bpw = jpw * BLK
    nk = S // 8
    upk = 8 * jpw
    hshift = (D - 1).bit_length()

    mesh = plsc.VectorSubcoreMesh(
        core_axis_name="c", subcore_axis_name="s",
        num_cores=NC, num_subcores=NS)

    @functools.partial(
        pl.kernel,
        mesh=mesh,
        out_type=jax.ShapeDtypeStruct((S, D, BT), jnp.float32),
        scratch_types=[
            pltpu.VMEM((8, bpw), jnp.int32),
            pltpu.VMEM((2, BLK), jnp.int32),
            pltpu.VMEM((2, BLK, 2 * D), jnp.float32),
            pltpu.VMEM((2, D, BLK), jnp.float32),
            pltpu.SemaphoreType.DMA,
            pltpu.SemaphoreType.DMA,
            pltpu.SemaphoreType.DMA,
            pltpu.SemaphoreType.DMA,
        ],
        compiler_params=pltpu.CompilerParams(
            use_tc_tiling_on_sc=True, needs_layout_passes=False),
    )
    def embed_kernel(tab_hbm, ids_hbm, out_hbm, ids_v, kidx_v, g_v, t_v,
                     gsem0, gsem1, osem0, osem1):
        gsem = (gsem0, gsem1)
        osem = (osem0, osem1)
        wid = lax.axis_index("s") * NC + lax.axis_index("c")
        b0 = wid * bpw
        iota16 = lax.iota(jnp.int32, L)
        skew = [(iota16 + i) & (L - 1) for i in range(L)]

        def fire_gather(u, q):
            r = u // jpw
            base = (u % jpw) * BLK
            for cc in range(BLK // L):
                ids16 = ids_v[r, pl.ds(base + cc * L, L)]
                kidx_v[q, pl.ds(cc * L, L)] = ids16 >> 1
            pltpu.async_copy(
                tab_hbm.at[kidx_v.at[q]], g_v.at[q], gsem[q])

        def gather_wait(q):
            pltpu.make_async_copy(
                tab_hbm.at[kidx_v.at[q]], g_v.at[q], gsem[q]).wait()

        def out_wait(q):
            pltpu.make_async_copy(
                t_v.at[q],
                out_hbm.at[0, pl.ds(0, D), pl.ds(0, BLK)], osem[q]).wait()

        def do_unit(k, u, p):
            s0 = k * 8
            r = u // jpw
            base = (u % jpw) * BLK

            @pl.when(u + 1 < upk)
            def _():
                fire_gather(u + 1, 1 - p)

            gather_wait(p)

            @pl.when(k * upk + u >= 2)
            def _():
                out_wait(p)

            gp = g_v.at[p]
            for jj in range(BLK // L):
                ids16 = ids_v[r, pl.ds(base + jj * L, L)]
                col0 = (ids16 & 1) << hshift
                rows = iota16 + (jj * L)
                scol = iota16 + (jj * L)

                def drow(d16, _, col0=col0, rows=rows, scol=scol):
                    dbase = d16 * L
                    for i in range(L):
                        dvec = skew[i] + dbase
                        vals = plsc.load_gather(gp, [rows, col0 + dvec])
                        plsc.store_scatter(t_v.at[p], [dvec, scol], vals)
                    return 0

                lax.fori_loop(0, D // 16, drow, 0)

            pltpu.async_copy(
                t_v.at[p],
                out_hbm.at[s0 + r, pl.ds(0, D), pl.ds(b0 + base, BLK)],
                osem[p])

        def kblock(k, _):
            pltpu.sync_copy(
                ids_hbm.at[pl.ds(k * 8, 8), pl.ds(b0, bpw)], ids_v)
            fire_gather(0, 0)

            def pair(g, _):
                do_unit(k, 2 * g, 0)
                do_unit(k, 2 * g + 1, 1)
                return 0

            lax.fori_loop(0, upk // 2, pair, 0)
            return 0

        lax.fori_loop(0, nk, kblock, 0)
        out_wait(0)
        out_wait(1)

    return embed_kernel


def kernel(token_ids, weight):
    BT, S = token_ids.shape
    V, D = weight.shape
    ids_t = token_ids.T.astype(jnp.int32)
    tab2 = weight.reshape(V // 2, 2 * D)
    out3 = _make_embed(S, BT, D)(tab2, ids_t)
    return out3.transpose(2, 0, 1)


# speedup vs baseline: 3.0362x; 1.6306x over previous
import functools

import jax
import jax.numpy as jnp
from jax import lax
from jax.experimental import pallas as pl
from jax.experimental.pallas import tpu as pltpu
from jax.experimental.pallas import tpu_sc as plsc

NC = 2
NS = 16
NW = NC * NS

IDXW = 128


@functools.lru_cache(maxsize=None)
def _make_gather(B, V, D, chunk):
    assert B % (NW * chunk) == 0 and chunk % IDXW == 0
    rows_per_w = B // NW
    steps = rows_per_w // chunk
    assert steps % 2 == 0
    gpc = chunk // IDXW

    mesh = plsc.VectorSubcoreMesh(
        core_axis_name="c", subcore_axis_name="s",
        num_cores=NC, num_subcores=NS)

    @functools.partial(
        pl.kernel,
        mesh=mesh,
        out_type=jax.ShapeDtypeStruct((B, 2 * D), jnp.float32),
        scratch_types=[
            pltpu.VMEM((2, chunk), jnp.int32),
            pltpu.VMEM((2, chunk, D), jnp.float32),
            pltpu.SemaphoreType.DMA,
            pltpu.SemaphoreType.DMA,
            pltpu.SemaphoreType.DMA,
            pltpu.SemaphoreType.DMA,
            pltpu.SemaphoreType.DMA,
            pltpu.SemaphoreType.DMA,
        ],
        compiler_params=pltpu.CompilerParams(use_tc_tiling_on_sc=False),
    )
    def gather_kernel(table_hbm, idx_hbm, out_hbm, idx_v, rows_v,
                      isem0, isem1, gsem0, gsem1, osem0, osem1):
        isem = (isem0, isem1)
        gsem = (gsem0, gsem1)
        osem = (osem0, osem1)
        wid = lax.axis_index("s") * NC + lax.axis_index("c")
        base = wid * rows_per_w

        def idx_fetch(s, p):
            pltpu.async_copy(
                idx_hbm.at[pl.ds(base + s * chunk, chunk)], idx_v.at[p],
                isem[p])

        def idx_wait(p):
            pltpu.make_async_copy(
                idx_hbm.at[pl.ds(0, chunk)], idx_v.at[p], isem[p]).wait()

        def out_wait(p):
            pltpu.make_async_copy(
                rows_v.at[p],
                out_hbm.at[pl.ds(0, chunk), pl.ds(0, D)], osem[p]).wait()

        def do_step(s, p):
            idx_wait(p)
            @pl.when(s >= 2)
            def _():
                out_wait(p)
            copies = [
                pltpu.async_copy(
                    table_hbm.at[idx_v.at[p, pl.ds(j * IDXW, IDXW)]],
                    rows_v.at[p, pl.ds(j * IDXW, IDXW)],
                    gsem[p],
                )
                for j in range(gpc)
            ]
            for c in copies:
                c.wait()

            @pl.when(s + 2 < steps)
            def _():
                idx_fetch(s + 2, p)

            pltpu.async_copy(
                rows_v.at[p],
                out_hbm.at[pl.ds(base + s * chunk, chunk), pl.ds(0, D)],
                osem[p])

        idx_fetch(0, 0)
        idx_fetch(1, 1)

        def body(g, _):
            do_step(2 * g, 0)
            do_step(2 * g + 1, 1)
            return 0

        lax.fori_loop(0, steps // 2, body, 0)
        out_wait(0)
        out_wait(1)

    return gather_kernel


def kernel(token_ids, weight):
    B = token_ids.size
    V, D = weight.shape
    idx = token_ids.reshape(B).astype(jnp.int32)
    out = _make_gather(B, V, D, 640)(weight, idx)
    return out[:, :D].reshape(*token_ids.shape, D)
